# baseline (speedup 1.0000x reference)
import functools

import jax
import jax.numpy as jnp
from jax import lax
from jax.experimental import pallas as pl
from jax.experimental.pallas import tpu as pltpu
from jax.experimental.pallas import tpu_sc as plsc

N = 10000
E = 320000
D = 128
B = 8

NC = 2
NS = 16
NW = NC * NS
EW = E // NW
CH = 128
NFULL = EW // CH
TAIL = EW - NFULL * CH
TRB = 624
TEX = N - NS * TRB
ZR = 104

RB = 1000
NRB = N // RB
EB = 512
NEB = E // EB


def _iota_oh(b):
    return (b[:, None] == lax.broadcasted_iota(jnp.int32, (1, B), 1)).astype(
        jnp.float32)


def _pre_body(x_ref, b3_ref, u_ref, wa_ref, wb_ref, wd_ref, be1_ref,
              p1_ref, p2_ref):
    x = x_ref[...]
    oh = _iota_oh(b3_ref[0, 0, :])
    u1 = jnp.dot(u_ref[...], wd_ref[...]) + be1_ref[...]
    p1_ref[...] = jnp.dot(x, wa_ref[...]) + jnp.dot(oh, u1)
    p2_ref[...] = jnp.dot(x, wb_ref[...])


def _precompute(x, batch3, u, wa, wb, wd, be1):
    return pl.pallas_call(
        _pre_body,
        grid=(NRB,),
        in_specs=[
            pl.BlockSpec((RB, D), lambda i: (i, 0)),
            pl.BlockSpec((1, 1, RB), lambda i: (i, 0, 0)),
            pl.BlockSpec((B, D), lambda i: (0, 0)),
            pl.BlockSpec((D, D), lambda i: (0, 0)),
            pl.BlockSpec((D, D), lambda i: (0, 0)),
            pl.BlockSpec((D, D), lambda i: (0, 0)),
            pl.BlockSpec((D,), lambda i: (0,)),
        ],
        out_specs=[
            pl.BlockSpec((RB, D), lambda i: (i, 0)),
            pl.BlockSpec((RB, D), lambda i: (i, 0)),
        ],
        out_shape=[
            jax.ShapeDtypeStruct((N, D), jnp.float32),
            jax.ShapeDtypeStruct((N, D), jnp.float32),
        ],
    )(x, batch3, u, wa, wb, wd, be1)


def _sc_gather(p1, p2, src, dst, batchv):
    mesh = plsc.VectorSubcoreMesh(core_axis_name="c", subcore_axis_name="s",
                                  num_cores=NC, num_subcores=NS)

    @functools.partial(
        pl.kernel,
        out_type=(
            jax.ShapeDtypeStruct((E, D), jnp.float32),
            jax.ShapeDtypeStruct((E,), jnp.int32),
        ),
        mesh=mesh,
        compiler_params=pltpu.CompilerParams(needs_layout_passes=False),
        scratch_types=[
            pltpu.VMEM((EW,), jnp.int32),
            pltpu.VMEM((EW,), jnp.int32),
            pltpu.VMEM((EW,), jnp.int32),
            pltpu.VMEM((N,), jnp.int32),
            pltpu.VMEM((2, CH, D), jnp.float32),
            pltpu.VMEM((2, CH, D), jnp.float32),
            pltpu.VMEM((TAIL, D), jnp.float32),
            pltpu.VMEM((TAIL, D), jnp.float32),
            pltpu.SemaphoreType.DMA,
            pltpu.SemaphoreType.DMA,
        ],
    )
    def k(p1_hbm, p2_hbm, src_hbm, dst_hbm, batch_hbm,
          g_hbm, bsrc_hbm,
          sidx_all, didx_all, bsrc_all, batch_v,
          rows_a, rows_b, rows_at, rows_bt, gsem, wsem):
        wid = lax.axis_index("s") * NC + lax.axis_index("c")
        wbase = wid * EW
        pltpu.sync_copy(src_hbm.at[pl.ds(wbase, EW)], sidx_all)
        pltpu.sync_copy(dst_hbm.at[pl.ds(wbase, EW)], didx_all)
        pltpu.sync_copy(batch_hbm, batch_v)

        def fire(j, buf):
            pltpu.async_copy(
                p1_hbm.at[sidx_all.at[pl.ds(j * CH, CH)]], rows_a.at[buf],
                gsem)
            pltpu.async_copy(
                p2_hbm.at[didx_all.at[pl.ds(j * CH, CH)]], rows_b.at[buf],
                gsem)

        fire(0, 0)

        def body(j, carry):
            p = j & 1
            q = 1 - p
            pltpu.make_async_copy(p1_hbm.at[sidx_all.at[pl.ds(0, CH)]],
                                  rows_a.at[p], gsem).wait()
            pltpu.make_async_copy(p1_hbm.at[sidx_all.at[pl.ds(0, CH)]],
                                  rows_b.at[p], gsem).wait()

            @pl.when(j >= 1)
            def _():
                pltpu.make_async_copy(rows_a.at[q],
                                      g_hbm.at[pl.ds(0, CH)], wsem).wait()

            @pl.when(j < NFULL - 1)
            def _():
                fire(j + 1, q)

            def vadd(r, c):
                for kk in range(D // 16):
                    sl = pl.ds(kk * 16, 16)
                    rows_a[p, r, sl] = rows_a[p, r, sl] + rows_b[p, r, sl]
                return c

            lax.fori_loop(0, CH, vadd, 0)
            for kk in range(CH // 16):
                off = pl.ds(j * CH + kk * 16, 16)
                bsrc_all[off] = plsc.load_gather(batch_v, [sidx_all[off]])
            pltpu.async_copy(rows_a.at[p],
                             g_hbm.at[pl.ds(wbase + j * CH, CH)], wsem)
            return carry

        lax.fori_loop(0, NFULL, body, 0)
        pltpu.make_async_copy(rows_a.at[(NFULL - 1) & 1],
                              g_hbm.at[pl.ds(0, CH)], wsem).wait()

        tb = NFULL * CH
        c1 = pltpu.async_copy(p1_hbm.at[sidx_all.at[pl.ds(tb, TAIL)]],
                              rows_at, gsem)
        c2 = pltpu.async_copy(p2_hbm.at[didx_all.at[pl.ds(tb, TAIL)]],
                              rows_bt, gsem)
        c1.wait()
        c2.wait()

        def vaddt(r, c):
            for kk in range(D // 16):
                sl = pl.ds(kk * 16, 16)
                rows_at[r, sl] = rows_at[r, sl] + rows_bt[r, sl]
            return c

        lax.fori_loop(0, TAIL, vaddt, 0)
        off = pl.ds(tb, TAIL)
        bsrc_all[off] = plsc.load_gather(batch_v, [sidx_all[off]])
        pltpu.sync_copy(rows_at, g_hbm.at[pl.ds(wbase + tb, TAIL)])
        pltpu.sync_copy(bsrc_all, bsrc_hbm.at[pl.ds(wbase, EW)])

    return k(p1, p2, src, dst, batchv)


def _edge_body(ea_ref, g_ref, b3_ref, wc_ref, w2_ref, be2_ref,
               en_ref, eg_ref, acc_ref):
    i = pl.program_id(0)
    ea = ea_ref[...]
    h = jnp.maximum(g_ref[...] + jnp.dot(ea, wc_ref[...]), 0.0)
    en = ea + jnp.dot(h, w2_ref[...]) + be2_ref[...]
    en_ref[...] = en
    oh = _iota_oh(b3_ref[0, 0, :])

    @pl.when(i == 0)
    def _():
        acc_ref[...] = jnp.zeros_like(acc_ref)

    acc_ref[...] += lax.dot_general(oh, en, (((0,), (0,)), ((), ())))

    @pl.when(i == NEB - 1)
    def _():
        eg_ref[...] = acc_ref[...]


def _edge_mlp(edge_attr, g, bsrc3, wc, w2, be2):
    return pl.pallas_call(
        _edge_body,
        grid=(NEB,),
        in_specs=[
            pl.BlockSpec((EB, D), lambda i: (i, 0)),
            pl.BlockSpec((EB, D), lambda i: (i, 0)),
            pl.BlockSpec((1, 1, EB), lambda i: (i, 0, 0)),
            pl.BlockSpec((D, D), lambda i: (0, 0)),
            pl.BlockSpec((D, D), lambda i: (0, 0)),
            pl.BlockSpec((D,), lambda i: (0,)),
        ],
        out_specs=[
            pl.BlockSpec((EB, D), lambda i: (i, 0)),
            pl.BlockSpec((B, D), lambda i: (0, 0)),
        ],
        out_shape=[
            jax.ShapeDtypeStruct((E, D), jnp.float32),
            jax.ShapeDtypeStruct((B, D), jnp.float32),
        ],
        scratch_shapes=[pltpu.VMEM((B, D), jnp.float32)],
    )(edge_attr, g, bsrc3, wc, w2, be2)


def _sc_scatter(e_new, dst):
    mesh = plsc.VectorSubcoreMesh(core_axis_name="c", subcore_axis_name="s",
                                  num_cores=NC, num_subcores=NS)

    @functools.partial(
        pl.kernel,
        out_type=jax.ShapeDtypeStruct((NC * N, D), jnp.float32),
        mesh=mesh,
        compiler_params=pltpu.CompilerParams(needs_layout_passes=False),
        scratch_types=[
            pltpu.VMEM_SHARED((N, D), jnp.float32),
            pltpu.VMEM((2, CH), jnp.int32),
            pltpu.VMEM((2, CH, D), jnp.float32),
            pltpu.VMEM((TAIL,), jnp.int32),
            pltpu.VMEM((TAIL, D), jnp.float32),
            pltpu.VMEM((ZR, D), jnp.float32),
            pltpu.SemaphoreType.DMA,
            pltpu.SemaphoreType.DMA,
        ],
    )
    def k(en_hbm, dst_hbm, out_hbm,
          spmem, didx, rows, didx_t, rows_t, zbuf, isem, rsem):
        cid = lax.axis_index("c")
        sid = lax.axis_index("s")
        wid = sid * NC + cid

        def zb(r, carry):
            for kk in range(D // 16):
                zbuf[r, pl.ds(kk * 16, 16)] = jnp.zeros((16,), jnp.float32)
            return carry

        lax.fori_loop(0, ZR, zb, 0)
        tbase = sid * TRB
        for kk in range(TRB // ZR):
            pltpu.sync_copy(zbuf, spmem.at[pl.ds(tbase + kk * ZR, ZR)])

        @pl.when(sid == 0)
        def _():
            pltpu.sync_copy(zbuf.at[pl.ds(0, TEX)],
                            spmem.at[pl.ds(NS * TRB, TEX)])

        plsc.subcore_barrier()

        def fire(j, buf):
            base = wid * EW + j * CH
            pltpu.async_copy(dst_hbm.at[pl.ds(base, CH)], didx.at[buf], isem)
            pltpu.async_copy(en_hbm.at[pl.ds(base, CH)], rows.at[buf], rsem)

        fire(0, 0)

        def body(j, carry):
            p = j & 1
            q = 1 - p
            pltpu.make_async_copy(dst_hbm.at[pl.ds(0, CH)],
                                  didx.at[p], isem).wait()
            pltpu.make_async_copy(en_hbm.at[pl.ds(0, CH)],
                                  rows.at[p], rsem).wait()

            @pl.when(j < NFULL - 1)
            def _():
                fire(j + 1, q)

            pltpu.sync_copy(rows.at[p], spmem.at[didx.at[p]], add=True)
            return carry

        lax.fori_loop(0, NFULL, body, 0)
        tb = wid * EW + NFULL * CH
        pltpu.sync_copy(dst_hbm.at[pl.ds(tb, TAIL)], didx_t)
        pltpu.sync_copy(en_hbm.at[pl.ds(tb, TAIL)], rows_t)
        pltpu.sync_copy(rows_t, spmem.at[didx_t], add=True)
        plsc.subcore_barrier()
        pltpu.sync_copy(spmem.at[pl.ds(tbase, TRB)],
                        out_hbm.at[pl.ds(cid * N + tbase, TRB)])

        @pl.when(sid == 0)
        def _():
            pltpu.sync_copy(spmem.at[pl.ds(NS * TRB, TEX)],
                            out_hbm.at[pl.ds(cid * N + NS * TRB, TEX)])

    return k(e_new, dst)


def _node_body(x_ref, aa_ref, ab_ref, b3_ref, u_ref, eg_ref,
               wna_ref, wnb_ref, wnc_ref, bn1_ref, wn2_ref, bn2_ref,
               wga_ref, wgb_ref, wgc_ref, bg1_ref, wg2_ref, bg2_ref,
               xn_ref, un_ref, acc_ref):
    i = pl.program_id(0)
    x = x_ref[...]
    agg = aa_ref[...] + ab_ref[...]
    oh = _iota_oh(b3_ref[0, 0, :])
    u = u_ref[...]
    u3 = jnp.dot(u, wnc_ref[...])
    h = jnp.maximum(
        jnp.dot(x, wna_ref[...]) + jnp.dot(agg, wnb_ref[...])
        + jnp.dot(oh, u3) + bn1_ref[...], 0.0)
    xn = x + jnp.dot(h, wn2_ref[...]) + bn2_ref[...]
    xn_ref[...] = xn

    @pl.when(i == 0)
    def _():
        acc_ref[...] = jnp.zeros_like(acc_ref)

    acc_ref[...] += lax.dot_general(oh, xn, (((0,), (0,)), ((), ())))

    @pl.when(i == NRB - 1)
    def _():
        ng = acc_ref[...]
        g = jnp.maximum(
            jnp.dot(ng, wga_ref[...]) + jnp.dot(eg_ref[...], wgb_ref[...])
            + jnp.dot(u, wgc_ref[...]) + bg1_ref[...], 0.0)
        un_ref[...] = u + jnp.dot(g, wg2_ref[...]) + bg2_ref[...]


def _node_global(x, aggp, batch3, u, eg,
                 wna, wnb, wnc, bn1, wn2, bn2,
                 wga, wgb, wgc, bg1, wg2, bg2):
    wspec = pl.BlockSpec((D, D), lambda i: (0, 0))
    bspec = pl.BlockSpec((D,), lambda i: (0,))
    return pl.pallas_call(
        _node_body,
        grid=(NRB,),
        in_specs=[
            pl.BlockSpec((RB, D), lambda i: (i, 0)),
            pl.BlockSpec((RB, D), lambda i: (i, 0)),
            pl.BlockSpec((RB, D), lambda i: (i + NRB, 0)),
            pl.BlockSpec((1, 1, RB), lambda i: (i, 0, 0)),
            pl.BlockSpec((B, D), lambda i: (0, 0)),
            pl.BlockSpec((B, D), lambda i: (0, 0)),
            wspec, wspec, wspec, bspec, wspec, bspec,
            wspec, wspec, wspec, bspec, wspec, bspec,
        ],
        out_specs=[
            pl.BlockSpec((RB, D), lambda i: (i, 0)),
            pl.BlockSpec((B, D), lambda i: (0, 0)),
        ],
        out_shape=[
            jax.ShapeDtypeStruct((N, D), jnp.float32),
            jax.ShapeDtypeStruct((B, D), jnp.float32),
        ],
        scratch_shapes=[pltpu.VMEM((B, D), jnp.float32)],
    )(x, aggp, aggp, batch3, u, eg,
      wna, wnb, wnc, bn1, wn2, bn2,
      wga, wgb, wgc, bg1, wg2, bg2)


def kernel(x, edge_attr, u, edge_index, batch,
           We1, be1, We2, be2,
           Wn1, bn1, Wn2, bn2,
           Wg1, bg1, Wg2, bg2):
    src = edge_index[0].astype(jnp.int32)
    dst = edge_index[1].astype(jnp.int32)
    batch32 = batch.astype(jnp.int32)
    batch3 = batch32.reshape(NRB, 1, RB)

    p1, p2 = _precompute(x, batch3, u, We1[:D], We1[D:2 * D], We1[3 * D:],
                         be1)
    g, bsrc = _sc_gather(p1, p2, src, dst, batch32)
    e_new, edge_g = _edge_mlp(edge_attr, g,
                              bsrc.reshape(NEB, 1, EB),
                              We1[2 * D:3 * D], We2, be2)
    aggp = _sc_scatter(e_new, dst)
    x_new, u_new = _node_global(
        x, aggp, batch3, u, edge_g,
        Wn1[:D], Wn1[D:2 * D], Wn1[2 * D:], bn1, Wn2, bn2,
        Wg1[:D], Wg1[D:2 * D], Wg1[2 * D:], bg1, Wg2, bg2)
    return (x_new, e_new, u_new)


# speedup vs baseline: 1.2477x; 1.2477x over previous
import functools

import jax
import jax.numpy as jnp
from jax import lax
from jax.experimental import pallas as pl
from jax.experimental.pallas import tpu as pltpu
from jax.experimental.pallas import tpu_sc as plsc

N = 10000
E = 320000
D = 128
B = 8

NC = 2
NS = 16
NW = NC * NS
EW = E // NW
CH = 128
NFULL = EW // CH
TAIL = EW - NFULL * CH
TRB = 624
TEX = N - NS * TRB
ZR = 104

RB = 1000
NRB = N // RB
EB = 512
NEB = E // EB


def _iota_oh(b):
    return (b[:, None] == lax.broadcasted_iota(jnp.int32, (1, B), 1)).astype(
        jnp.float32)


def _pre_body(x_ref, b3_ref, u_ref, wa_ref, wb_ref, wd_ref, be1_ref,
              p1_ref, p2_ref):
    x = x_ref[...]
    oh = _iota_oh(b3_ref[0, 0, :])
    u1 = jnp.dot(u_ref[...], wd_ref[...]) + be1_ref[...]
    p1_ref[...] = jnp.dot(x, wa_ref[...]) + jnp.dot(oh, u1)
    p2_ref[...] = jnp.dot(x, wb_ref[...])


def _precompute(x, batch3, u, wa, wb, wd, be1):
    return pl.pallas_call(
        _pre_body,
        grid=(NRB,),
        in_specs=[
            pl.BlockSpec((RB, D), lambda i: (i, 0)),
            pl.BlockSpec((1, 1, RB), lambda i: (i, 0, 0)),
            pl.BlockSpec((B, D), lambda i: (0, 0)),
            pl.BlockSpec((D, D), lambda i: (0, 0)),
            pl.BlockSpec((D, D), lambda i: (0, 0)),
            pl.BlockSpec((D, D), lambda i: (0, 0)),
            pl.BlockSpec((D,), lambda i: (0,)),
        ],
        out_specs=[
            pl.BlockSpec((RB, D), lambda i: (i, 0)),
            pl.BlockSpec((RB, D), lambda i: (i, 0)),
        ],
        out_shape=[
            jax.ShapeDtypeStruct((N, D), jnp.float32),
            jax.ShapeDtypeStruct((N, D), jnp.float32),
        ],
    )(x, batch3, u, wa, wb, wd, be1)


def _sc_gather(p1, p2, src, dst, batchv):
    mesh = plsc.VectorSubcoreMesh(core_axis_name="c", subcore_axis_name="s",
                                  num_cores=NC, num_subcores=NS)

    @functools.partial(
        pl.kernel,
        out_type=(
            jax.ShapeDtypeStruct((E, D), jnp.float32),
            jax.ShapeDtypeStruct((E, D), jnp.float32),
            jax.ShapeDtypeStruct((E,), jnp.int32),
        ),
        mesh=mesh,
        compiler_params=pltpu.CompilerParams(needs_layout_passes=False),
        scratch_types=[
            pltpu.VMEM((EW,), jnp.int32),
            pltpu.VMEM((EW,), jnp.int32),
            pltpu.VMEM((EW,), jnp.int32),
            pltpu.VMEM((N,), jnp.int32),
            pltpu.VMEM((2, CH, D), jnp.float32),
            pltpu.VMEM((2, CH, D), jnp.float32),
            pltpu.VMEM((TAIL, D), jnp.float32),
            pltpu.VMEM((TAIL, D), jnp.float32),
            pltpu.SemaphoreType.DMA,
            pltpu.SemaphoreType.DMA,
        ],
    )
    def k(p1_hbm, p2_hbm, src_hbm, dst_hbm, batch_hbm,
          gs_hbm, gd_hbm, bsrc_hbm,
          sidx_all, didx_all, bsrc_all, batch_v,
          rows_a, rows_b, rows_at, rows_bt, gsem, wsem):
        wid = lax.axis_index("s") * NC + lax.axis_index("c")
        wbase = wid * EW
        pltpu.sync_copy(src_hbm.at[pl.ds(wbase, EW)], sidx_all)
        pltpu.sync_copy(dst_hbm.at[pl.ds(wbase, EW)], didx_all)
        pltpu.sync_copy(batch_hbm, batch_v)

        def fire(j, buf):
            pltpu.async_copy(
                p1_hbm.at[sidx_all.at[pl.ds(j * CH, CH)]], rows_a.at[buf],
                gsem)
            pltpu.async_copy(
                p2_hbm.at[didx_all.at[pl.ds(j * CH, CH)]], rows_b.at[buf],
                gsem)

        fire(0, 0)

        def body(j, carry):
            p = j & 1
            q = 1 - p
            pltpu.make_async_copy(p1_hbm.at[sidx_all.at[pl.ds(0, CH)]],
                                  rows_a.at[p], gsem).wait()
            pltpu.make_async_copy(p1_hbm.at[sidx_all.at[pl.ds(0, CH)]],
                                  rows_b.at[p], gsem).wait()

            @pl.when(j >= 1)
            def _():
                pltpu.make_async_copy(rows_a.at[q],
                                      gs_hbm.at[pl.ds(0, CH)], wsem).wait()
                pltpu.make_async_copy(rows_b.at[q],
                                      gd_hbm.at[pl.ds(0, CH)], wsem).wait()

            @pl.when(j < NFULL - 1)
            def _():
                fire(j + 1, q)

            for kk in range(CH // 16):
                off = pl.ds(j * CH + kk * 16, 16)
                bsrc_all[off] = plsc.load_gather(batch_v, [sidx_all[off]])
            pltpu.async_copy(rows_a.at[p],
                             gs_hbm.at[pl.ds(wbase + j * CH, CH)], wsem)
            pltpu.async_copy(rows_b.at[p],
                             gd_hbm.at[pl.ds(wbase + j * CH, CH)], wsem)
            return carry

        lax.fori_loop(0, NFULL, body, 0)
        pltpu.make_async_copy(rows_a.at[(NFULL - 1) & 1],
                              gs_hbm.at[pl.ds(0, CH)], wsem).wait()
        pltpu.make_async_copy(rows_b.at[(NFULL - 1) & 1],
                              gd_hbm.at[pl.ds(0, CH)], wsem).wait()

        tb = NFULL * CH
        c1 = pltpu.async_copy(p1_hbm.at[sidx_all.at[pl.ds(tb, TAIL)]],
                              rows_at, gsem)
        c2 = pltpu.async_copy(p2_hbm.at[didx_all.at[pl.ds(tb, TAIL)]],
                              rows_bt, gsem)
        c1.wait()
        c2.wait()
        off = pl.ds(tb, TAIL)
        bsrc_all[off] = plsc.load_gather(batch_v, [sidx_all[off]])
        pltpu.sync_copy(rows_at, gs_hbm.at[pl.ds(wbase + tb, TAIL)])
        pltpu.sync_copy(rows_bt, gd_hbm.at[pl.ds(wbase + tb, TAIL)])
        pltpu.sync_copy(bsrc_all, bsrc_hbm.at[pl.ds(wbase, EW)])

    return k(p1, p2, src, dst, batchv)


def _edge_body(ea_ref, gs_ref, gd_ref, b3_ref, wc_ref, w2_ref, be2_ref,
               en_ref, eg_ref, acc_ref):
    i = pl.program_id(0)
    ea = ea_ref[...]
    h = jnp.maximum(gs_ref[...] + gd_ref[...] + jnp.dot(ea, wc_ref[...]), 0.0)
    en = ea + jnp.dot(h, w2_ref[...]) + be2_ref[...]
    en_ref[...] = en
    oh = _iota_oh(b3_ref[0, 0, :])

    @pl.when(i == 0)
    def _():
        acc_ref[...] = jnp.zeros_like(acc_ref)

    acc_ref[...] += lax.dot_general(oh, en, (((0,), (0,)), ((), ())))

    @pl.when(i == NEB - 1)
    def _():
        eg_ref[...] = acc_ref[...]


def _edge_mlp(edge_attr, gs, gd, bsrc3, wc, w2, be2):
    return pl.pallas_call(
        _edge_body,
        grid=(NEB,),
        in_specs=[
            pl.BlockSpec((EB, D), lambda i: (i, 0)),
            pl.BlockSpec((EB, D), lambda i: (i, 0)),
            pl.BlockSpec((EB, D), lambda i: (i, 0)),
            pl.BlockSpec((1, 1, EB), lambda i: (i, 0, 0)),
            pl.BlockSpec((D, D), lambda i: (0, 0)),
            pl.BlockSpec((D, D), lambda i: (0, 0)),
            pl.BlockSpec((D,), lambda i: (0,)),
        ],
        out_specs=[
            pl.BlockSpec((EB, D), lambda i: (i, 0)),
            pl.BlockSpec((B, D), lambda i: (0, 0)),
        ],
        out_shape=[
            jax.ShapeDtypeStruct((E, D), jnp.float32),
            jax.ShapeDtypeStruct((B, D), jnp.float32),
        ],
        scratch_shapes=[pltpu.VMEM((B, D), jnp.float32)],
    )(edge_attr, gs, gd, bsrc3, wc, w2, be2)


def _sc_scatter(e_new, dst):
    mesh = plsc.VectorSubcoreMesh(core_axis_name="c", subcore_axis_name="s",
                                  num_cores=NC, num_subcores=NS)

    @functools.partial(
        pl.kernel,
        out_type=jax.ShapeDtypeStruct((NC * N, D), jnp.float32),
        mesh=mesh,
        compiler_params=pltpu.CompilerParams(needs_layout_passes=False),
        scratch_types=[
            pltpu.VMEM_SHARED((N, D), jnp.float32),
            pltpu.VMEM((2, CH), jnp.int32),
            pltpu.VMEM((2, CH, D), jnp.float32),
            pltpu.VMEM((TAIL,), jnp.int32),
            pltpu.VMEM((TAIL, D), jnp.float32),
            pltpu.VMEM((ZR, D), jnp.float32),
            pltpu.SemaphoreType.DMA,
            pltpu.SemaphoreType.DMA,
        ],
    )
    def k(en_hbm, dst_hbm, out_hbm,
          spmem, didx, rows, didx_t, rows_t, zbuf, isem, rsem):
        cid = lax.axis_index("c")
        sid = lax.axis_index("s")
        wid = sid * NC + cid

        def zb(r, carry):
            for kk in range(D // 16):
                zbuf[r, pl.ds(kk * 16, 16)] = jnp.zeros((16,), jnp.float32)
            return carry

        lax.fori_loop(0, ZR, zb, 0)
        tbase = sid * TRB
        for kk in range(TRB // ZR):
            pltpu.sync_copy(zbuf, spmem.at[pl.ds(tbase + kk * ZR, ZR)])

        @pl.when(sid == 0)
        def _():
            pltpu.sync_copy(zbuf.at[pl.ds(0, TEX)],
                            spmem.at[pl.ds(NS * TRB, TEX)])

        plsc.subcore_barrier()

        def fire(j, buf):
            base = wid * EW + j * CH
            pltpu.async_copy(dst_hbm.at[pl.ds(base, CH)], didx.at[buf], isem)
            pltpu.async_copy(en_hbm.at[pl.ds(base, CH)], rows.at[buf], rsem)

        fire(0, 0)

        def body(j, carry):
            p = j & 1
            q = 1 - p
            pltpu.make_async_copy(dst_hbm.at[pl.ds(0, CH)],
                                  didx.at[p], isem).wait()
            pltpu.make_async_copy(en_hbm.at[pl.ds(0, CH)],
                                  rows.at[p], rsem).wait()

            @pl.when(j < NFULL - 1)
            def _():
                fire(j + 1, q)

            pltpu.sync_copy(rows.at[p], spmem.at[didx.at[p]], add=True)
            return carry

        lax.fori_loop(0, NFULL, body, 0)
        tb = wid * EW + NFULL * CH
        pltpu.sync_copy(dst_hbm.at[pl.ds(tb, TAIL)], didx_t)
        pltpu.sync_copy(en_hbm.at[pl.ds(tb, TAIL)], rows_t)
        pltpu.sync_copy(rows_t, spmem.at[didx_t], add=True)
        plsc.subcore_barrier()
        pltpu.sync_copy(spmem.at[pl.ds(tbase, TRB)],
                        out_hbm.at[pl.ds(cid * N + tbase, TRB)])

        @pl.when(sid == 0)
        def _():
            pltpu.sync_copy(spmem.at[pl.ds(NS * TRB, TEX)],
                            out_hbm.at[pl.ds(cid * N + NS * TRB, TEX)])

    return k(e_new, dst)


def _node_body(x_ref, aa_ref, ab_ref, b3_ref, u_ref, eg_ref,
               wna_ref, wnb_ref, wnc_ref, bn1_ref, wn2_ref, bn2_ref,
               wga_ref, wgb_ref, wgc_ref, bg1_ref, wg2_ref, bg2_ref,
               xn_ref, un_ref, acc_ref):
    i = pl.program_id(0)
    x = x_ref[...]
    agg = aa_ref[...] + ab_ref[...]
    oh = _iota_oh(b3_ref[0, 0, :])
    u = u_ref[...]
    u3 = jnp.dot(u, wnc_ref[...])
    h = jnp.maximum(
        jnp.dot(x, wna_ref[...]) + jnp.dot(agg, wnb_ref[...])
        + jnp.dot(oh, u3) + bn1_ref[...], 0.0)
    xn = x + jnp.dot(h, wn2_ref[...]) + bn2_ref[...]
    xn_ref[...] = xn

    @pl.when(i == 0)
    def _():
        acc_ref[...] = jnp.zeros_like(acc_ref)

    acc_ref[...] += lax.dot_general(oh, xn, (((0,), (0,)), ((), ())))

    @pl.when(i == NRB - 1)
    def _():
        ng = acc_ref[...]
        g = jnp.maximum(
            jnp.dot(ng, wga_ref[...]) + jnp.dot(eg_ref[...], wgb_ref[...])
            + jnp.dot(u, wgc_ref[...]) + bg1_ref[...], 0.0)
        un_ref[...] = u + jnp.dot(g, wg2_ref[...]) + bg2_ref[...]


def _node_global(x, aggp, batch3, u, eg,
                 wna, wnb, wnc, bn1, wn2, bn2,
                 wga, wgb, wgc, bg1, wg2, bg2):
    wspec = pl.BlockSpec((D, D), lambda i: (0, 0))
    bspec = pl.BlockSpec((D,), lambda i: (0,))
    return pl.pallas_call(
        _node_body,
        grid=(NRB,),
        in_specs=[
            pl.BlockSpec((RB, D), lambda i: (i, 0)),
            pl.BlockSpec((RB, D), lambda i: (i, 0)),
            pl.BlockSpec((RB, D), lambda i: (i + NRB, 0)),
            pl.BlockSpec((1, 1, RB), lambda i: (i, 0, 0)),
            pl.BlockSpec((B, D), lambda i: (0, 0)),
            pl.BlockSpec((B, D), lambda i: (0, 0)),
            wspec, wspec, wspec, bspec, wspec, bspec,
            wspec, wspec, wspec, bspec, wspec, bspec,
        ],
        out_specs=[
            pl.BlockSpec((RB, D), lambda i: (i, 0)),
            pl.BlockSpec((B, D), lambda i: (0, 0)),
        ],
        out_shape=[
            jax.ShapeDtypeStruct((N, D), jnp.float32),
            jax.ShapeDtypeStruct((B, D), jnp.float32),
        ],
        scratch_shapes=[pltpu.VMEM((B, D), jnp.float32)],
    )(x, aggp, aggp, batch3, u, eg,
      wna, wnb, wnc, bn1, wn2, bn2,
      wga, wgb, wgc, bg1, wg2, bg2)


def kernel(x, edge_attr, u, edge_index, batch,
           We1, be1, We2, be2,
           Wn1, bn1, Wn2, bn2,
           Wg1, bg1, Wg2, bg2):
    src = edge_index[0].astype(jnp.int32)
    dst = edge_index[1].astype(jnp.int32)
    batch32 = batch.astype(jnp.int32)
    batch3 = batch32.reshape(NRB, 1, RB)

    p1, p2 = _precompute(x, batch3, u, We1[:D], We1[D:2 * D], We1[3 * D:],
                         be1)
    gs, gd, bsrc = _sc_gather(p1, p2, src, dst, batch32)
    e_new, edge_g = _edge_mlp(edge_attr, gs, gd,
                              bsrc.reshape(NEB, 1, EB),
                              We1[2 * D:3 * D], We2, be2)
    aggp = _sc_scatter(e_new, dst)
    x_new, u_new = _node_global(
        x, aggp, batch3, u, edge_g,
        Wn1[:D], Wn1[D:2 * D], Wn1[2 * D:], bn1, Wn2, bn2,
        Wg1[:D], Wg1[D:2 * D], Wg1[2 * D:], bg1, Wg2, bg2)
    return (x_new, e_new, u_new)
